# Initial kernel scaffold; baseline (speedup 1.0000x reference)
import functools

import jax
import jax.numpy as jnp
from jax import lax
from jax.experimental import pallas as pl
from jax.experimental.pallas import tpu as pltpu
from jax.experimental.pallas import tpu_sc as plsc

N_USERS = 4096
N_ITEMS = 16384
N_ENTITY = 100000
N_REL = 32
DIM = 64
B = 4096
T = 32
NL = 2
NNZ = 655360
C_TEMP = 0.2
LGCN_LAYERS = 3
N_ALL = N_USERS + N_ITEMS

NC = 2
NS = 16
NW = NC * NS
LANES = 16

_SC_MESH = plsc.VectorSubcoreMesh(core_axis_name="c", subcore_axis_name="s")


_EPT = NNZ // NW
_CE = 512
_NCHUNK = _EPT // _CE
_RPT = N_ALL // NS
_ZROWS = 128


def _spmm_body(x_hbm, col_hbm, row_hbm, val_hbm, out_hbm,
               col_v, row_v, val_v, rows_v, zero_v, acc_sh, sem):
    cid = lax.axis_index("c")
    sid = lax.axis_index("s")
    wid = cid * NS + sid

    def _zb(i, carry):
        for j in range(DIM // LANES):
            zero_v[i, pl.ds(j * LANES, LANES)] = jnp.zeros((LANES,), jnp.float32)
        return carry
    lax.fori_loop(0, _ZROWS, _zb, 0)
    for k in range(_RPT // _ZROWS):
        pltpu.sync_copy(zero_v, acc_sh.at[pl.ds(sid * _RPT + k * _ZROWS, _ZROWS)])
    plsc.subcore_barrier()

    base_edge = wid * _EPT

    def _chunk(ci, carry):
        eb = base_edge + ci * _CE
        pltpu.sync_copy(col_hbm.at[pl.ds(eb, _CE)], col_v)
        pltpu.sync_copy(row_hbm.at[pl.ds(eb, _CE)], row_v)
        pltpu.sync_copy(val_hbm.at[pl.ds(eb, _CE)], val_v)
        pltpu.async_copy(x_hbm.at[col_v], rows_v, sem).wait()

        def _scale(e, c2):
            v = val_v[e]
            for j in range(DIM // LANES):
                sl = pl.ds(j * LANES, LANES)
                rows_v[e, sl] = rows_v[e, sl] * v
            return c2
        lax.fori_loop(0, _CE, _scale, 0, unroll=4)
        pltpu.sync_copy(rows_v, acc_sh.at[row_v], add=True)
        return carry
    lax.fori_loop(0, _NCHUNK, _chunk, 0)

    plsc.subcore_barrier()
    pltpu.sync_copy(acc_sh.at[pl.ds(sid * _RPT, _RPT)],
                    out_hbm.at[cid, pl.ds(sid * _RPT, _RPT)])


_spmm_call = pl.kernel(
    _spmm_body,
    out_type=jax.ShapeDtypeStruct((NC, N_ALL, DIM), jnp.float32),
    mesh=_SC_MESH,
    scratch_types=[
        pltpu.VMEM((_CE,), jnp.int32),
        pltpu.VMEM((_CE,), jnp.int32),
        pltpu.VMEM((_CE,), jnp.float32),
        pltpu.VMEM((_CE, DIM), jnp.float32),
        pltpu.VMEM((_ZROWS, DIM), jnp.float32),
        pltpu.VMEM_SHARED((N_ALL, DIM), jnp.float32),
        pltpu.SemaphoreType.DMA,
    ],
)


def _make_gather(n_rows, chunk):
    rpt = n_rows // NW
    nchunk = rpt // chunk
    assert rpt % chunk == 0 and chunk % 8 == 0

    def _body(table_hbm, idx_hbm, out_hbm, idx_v, rows_v, sem):
        wid = lax.axis_index("c") * NS + lax.axis_index("s")
        base = wid * rpt

        def _ck(ci, carry):
            rb = base + ci * chunk
            pltpu.sync_copy(idx_hbm.at[pl.ds(rb, chunk)], idx_v)
            pltpu.async_copy(table_hbm.at[idx_v], rows_v, sem).wait()
            pltpu.sync_copy(rows_v, out_hbm.at[pl.ds(rb, chunk)])
            return carry
        lax.fori_loop(0, nchunk, _ck, 0)

    return pl.kernel(
        _body,
        out_type=jax.ShapeDtypeStruct((n_rows, DIM), jnp.float32),
        mesh=_SC_MESH,
        scratch_types=[
            pltpu.VMEM((chunk,), jnp.int32),
            pltpu.VMEM((chunk, DIM), jnp.float32),
            pltpu.SemaphoreType.DMA,
        ],
    )


_EGATHER_ROWS = 671744
_entity_gather = _make_gather(_EGATHER_ROWS, 512)
_batch_gather = _make_gather(2 * B, 256)


_BG = 512


def _att_kernel(h_ref, r_ref, t_ref, rel_ref, w1_ref, w2_ref, w3_ref,
                att_ref, hmean_ref):
    h3 = h_ref[...]
    t3 = t_ref[...]
    r2 = r_ref[...]
    w1a = w1_ref[0:DIM, :]
    w1b = w1_ref[DIM:2 * DIM, :]
    rproj_tab = jnp.dot(rel_ref[...], w1b, preferred_element_type=jnp.float32)
    oh3 = (r2[:, :, None] ==
           lax.broadcasted_iota(jnp.int32, (1, 1, N_REL), 2)).astype(jnp.float32)
    oh2 = oh3.reshape(_BG * T, N_REL)
    rproj = jnp.dot(oh2, rproj_tab, preferred_element_type=jnp.float32)
    h2 = h3.reshape(_BG * T, DIM)
    x = jnp.maximum(jnp.dot(h2, w1a, preferred_element_type=jnp.float32) + rproj, 0.0)
    x = jnp.maximum(jnp.dot(x, w2_ref[...], preferred_element_type=jnp.float32), 0.0)
    x3 = x.reshape(_BG, T, DIM)
    w3row = w3_ref[...].reshape(1, DIM)
    cols = []
    for t in range(T):
        st = jnp.sum(x3[:, t, :] * w3row, axis=1, keepdims=True)
        cols.append(st)
    s = jnp.concatenate(cols, axis=1)
    w = jax.nn.sigmoid(s)
    w = jnp.exp(w)
    w = w / jnp.sum(w, axis=1, keepdims=True)
    acc = jnp.zeros((_BG, DIM), jnp.float32)
    hsum = jnp.zeros((_BG, DIM), jnp.float32)
    for t in range(T):
        acc = acc + w[:, t:t + 1] * t3[:, t, :]
        hsum = hsum + h3[:, t, :]
    att_ref[...] = acc
    hmean_ref[...] = hsum * (1.0 / T)


def _att_call(h3, r2, t3, rel, w1, w2, w3):
    grid = (B // _BG,)
    return pl.pallas_call(
        _att_kernel,
        grid=grid,
        in_specs=[
            pl.BlockSpec((_BG, T, DIM), lambda i: (i, 0, 0)),
            pl.BlockSpec((_BG, T), lambda i: (i, 0)),
            pl.BlockSpec((_BG, T, DIM), lambda i: (i, 0, 0)),
            pl.BlockSpec((N_REL, DIM), lambda i: (0, 0)),
            pl.BlockSpec((2 * DIM, DIM), lambda i: (0, 0)),
            pl.BlockSpec((DIM, DIM), lambda i: (0, 0)),
            pl.BlockSpec((DIM, 1), lambda i: (0, 0)),
        ],
        out_specs=[
            pl.BlockSpec((_BG, DIM), lambda i: (i, 0)),
            pl.BlockSpec((_BG, DIM), lambda i: (i, 0)),
        ],
        out_shape=[
            jax.ShapeDtypeStruct((B, DIM), jnp.float32),
            jax.ShapeDtypeStruct((B, DIM), jnp.float32),
        ],
    )(h3, r2, t3, rel, w1, w2, w3)


def _closs_kernel(a_ref, bfull_ref, bblk_ref, l_ref, l1_ref):
    i = pl.program_id(0)
    a = a_ref[...]
    bf = bfull_ref[...]
    bb = bblk_ref[...]
    an = a / (jnp.sqrt(jnp.sum(a * a, axis=1, keepdims=True)) + 1e-8)
    bn = bf / (jnp.sqrt(jnp.sum(bf * bf, axis=1, keepdims=True)) + 1e-8)
    bnb = bb / (jnp.sqrt(jnp.sum(bb * bb, axis=1, keepdims=True)) + 1e-8)
    logits = lax.dot_general(an, bn, (((1,), (1,)), ((), ())),
                             preferred_element_type=jnp.float32) * (1.0 / C_TEMP)
    m = jnp.max(logits, axis=1, keepdims=True)
    lse = jnp.log(jnp.sum(jnp.exp(logits - m), axis=1, keepdims=True)) + m
    pos = jnp.sum(an * bnb, axis=1, keepdims=True) * (1.0 / C_TEMP)
    lblk = jnp.sum(lse - pos)
    z = jnp.sum(a * bb, axis=1, keepdims=True)
    l1blk = jnp.sum(jnp.maximum(-z, 0.0) + jnp.log(1.0 + jnp.exp(-jnp.abs(z))))

    @pl.when(i == 0)
    def _init():
        l_ref[...] = jnp.zeros((1, 1), jnp.float32)
        l1_ref[...] = jnp.zeros((1, 1), jnp.float32)

    l_ref[...] = l_ref[...] + lblk
    l1_ref[...] = l1_ref[...] + l1blk


def _closs_call(a, b):
    grid = (B // _BG,)
    return pl.pallas_call(
        _closs_kernel,
        grid=grid,
        in_specs=[
            pl.BlockSpec((_BG, DIM), lambda i: (i, 0)),
            pl.BlockSpec((B, DIM), lambda i: (0, 0)),
            pl.BlockSpec((_BG, DIM), lambda i: (i, 0)),
        ],
        out_specs=[
            pl.BlockSpec((1, 1), lambda i: (0, 0)),
            pl.BlockSpec((1, 1), lambda i: (0, 0)),
        ],
        out_shape=[
            jax.ShapeDtypeStruct((1, 1), jnp.float32),
            jax.ShapeDtypeStruct((1, 1), jnp.float32),
        ],
    )(a, b, b)


def _assemble_kernel(g5_ref, u1b_ref, g6_ref, att0_ref, att1_ref,
                     hmean0_ref, i1b_ref, out_ref):
    g5 = g5_ref[...]
    usum = jnp.zeros((_BG, DIM), jnp.float32)
    for t in range(T):
        usum = usum + g5[:, t, :]
    e_u = usum * (1.0 / T) + u1b_ref[...]
    e_v = (g6_ref[...] + att0_ref[...] + att1_ref[...]
           + hmean0_ref[...] + i1b_ref[...])
    out_ref[...] = jax.nn.sigmoid(jnp.sum(e_u * e_v, axis=1, keepdims=True))


def _assemble_call(g5, u1b, g6, att0, att1, hmean0, i1b):
    grid = (B // _BG,)
    vec = pl.BlockSpec((_BG, DIM), lambda i: (i, 0))
    return pl.pallas_call(
        _assemble_kernel,
        grid=grid,
        in_specs=[
            pl.BlockSpec((_BG, T, DIM), lambda i: (i, 0, 0)),
            vec, vec, vec, vec, vec, vec,
        ],
        out_specs=pl.BlockSpec((_BG, 1), lambda i: (i, 0)),
        out_shape=jax.ShapeDtypeStruct((B, 1), jnp.float32),
    )(g5, u1b, g6, att0, att1, hmean0, i1b)


def _lgcn(x0, row, col, val):
    acc = x0
    ego = x0
    for _ in range(LGCN_LAYERS):
        p = _spmm_call(ego, col, row, val)
        ego = p[0] + p[1]
        acc = acc + ego
    return acc * (1.0 / (LGCN_LAYERS + 1))


def kernel(items, users, item_idx, user_h, user_r, user_t, item_h, item_r,
           item_t, entity_emb, relation_emb, all_embed, W1, W2, W3, adj_row,
           adj_col, adj_val, adj2_row, adj2_col, adj2_val, u_adjdency):
    items = items.astype(jnp.int32)
    users = users.astype(jnp.int32)
    item_idx = item_idx.astype(jnp.int32)
    user_h = user_h.astype(jnp.int32)
    item_h = item_h.astype(jnp.int32)
    item_r = item_r.astype(jnp.int32)
    item_t = item_t.astype(jnp.int32)
    adj_row = adj_row.astype(jnp.int32)
    adj_col = adj_col.astype(jnp.int32)
    adj2_row = adj2_row.astype(jnp.int32)
    adj2_col = adj2_col.astype(jnp.int32)

    y1 = _lgcn(all_embed, adj_row, adj_col, adj_val)
    y2 = _lgcn(all_embed, adj2_row, adj2_col, adj2_val)

    bidx = jnp.concatenate([users, N_USERS + item_idx])
    g_y1 = _batch_gather(y1, bidx)
    g_y2 = _batch_gather(y2, bidx)
    u1b, i1b = g_y1[:B], g_y1[B:]
    u2b, i2b = g_y2[:B], g_y2[B:]

    eidx = jnp.concatenate([
        item_h[0].reshape(-1), item_t[0].reshape(-1),
        item_h[1].reshape(-1), item_t[1].reshape(-1),
        user_h[0].reshape(-1), items,
        jnp.zeros((_EGATHER_ROWS - 5 * B * T - B,), jnp.int32),
    ])
    eg = _entity_gather(entity_emb, eidx)
    bt = B * T
    gh0 = eg[0 * bt:1 * bt].reshape(B, T, DIM)
    gt0 = eg[1 * bt:2 * bt].reshape(B, T, DIM)
    gh1 = eg[2 * bt:3 * bt].reshape(B, T, DIM)
    gt1 = eg[3 * bt:4 * bt].reshape(B, T, DIM)
    g5 = eg[4 * bt:5 * bt].reshape(B, T, DIM)
    g6 = eg[5 * bt:5 * bt + B]

    att0, hmean0 = _att_call(gh0, item_r[0], gt0, relation_emb, W1, W2, W3)
    att1, _ = _att_call(gh1, item_r[1], gt1, relation_emb, W1, W2, W3)

    lu, l1u = _closs_call(u1b, u2b)
    li, l1i = _closs_call(i1b, i2b)
    c_loss = ((lu[0, 0] + li[0, 0]) / (2.0 * B)
              + l1u[0, 0] / B + l1i[0, 0] / B)

    scores2 = _assemble_call(g5, u1b, g6, att0, att1, hmean0, i1b)
    return (scores2.reshape(B), c_loss)


# speedup vs baseline: 4.2121x; 4.2121x over previous
import functools

import jax
import jax.numpy as jnp
from jax import lax
from jax.experimental import pallas as pl
from jax.experimental.pallas import tpu as pltpu
from jax.experimental.pallas import tpu_sc as plsc

N_USERS = 4096
N_ITEMS = 16384
N_ENTITY = 100000
N_REL = 32
DIM = 64
B = 4096
T = 32
NL = 2
NNZ = 655360
C_TEMP = 0.2
LGCN_LAYERS = 3
N_ALL = N_USERS + N_ITEMS

NC = 2
NS = 16
NW = NC * NS
LANES = 16

_SC_MESH = plsc.VectorSubcoreMesh(
    core_axis_name="c", subcore_axis_name="s", num_cores=NC, num_subcores=NS)
_SC_PARAMS = pltpu.CompilerParams(use_tc_tiling_on_sc=False)


_EPT = NNZ // NW
_CE = 512
_NCHUNK = _EPT // _CE
_RPT = N_ALL // NS
_ZROWS = 128


def _spmm_body(x_hbm, col_hbm, row_hbm, val_hbm, out_hbm,
               col_v, row_v, val_v, rows_v, zero_v, acc_sh, sem):
    cid = lax.axis_index("c")
    sid = lax.axis_index("s")
    wid = cid * NS + sid

    def _zb(i, carry):
        for j in range(DIM // LANES):
            zero_v[i, pl.ds(j * LANES, LANES)] = jnp.zeros((LANES,), jnp.float32)
        return carry
    lax.fori_loop(0, _ZROWS, _zb, 0)
    for k in range(_RPT // _ZROWS):
        pltpu.sync_copy(zero_v, acc_sh.at[pl.ds(sid * _RPT + k * _ZROWS, _ZROWS)])
    plsc.subcore_barrier()

    base_edge = wid * _EPT

    def _chunk(ci, carry):
        eb = base_edge + ci * _CE
        pltpu.sync_copy(col_hbm.at[pl.ds(eb, _CE)], col_v)
        pltpu.sync_copy(row_hbm.at[pl.ds(eb, _CE)], row_v)
        pltpu.sync_copy(val_hbm.at[pl.ds(eb, _CE)], val_v)
        pltpu.async_copy(x_hbm.at[col_v], rows_v, sem).wait()

        def _scale(g, c2):
            vv = val_v[pl.ds(g * LANES, LANES)]
            for j in range(LANES):
                v = vv[j]
                e = g * LANES + j
                for k in range(DIM // LANES):
                    sl = pl.ds(k * LANES, LANES)
                    rows_v[e, sl] = rows_v[e, sl] * v
            return c2
        lax.fori_loop(0, _CE // LANES, _scale, 0)
        pltpu.sync_copy(rows_v, acc_sh.at[row_v], add=True)
        return carry
    lax.fori_loop(0, _NCHUNK, _chunk, 0)

    plsc.subcore_barrier()
    pltpu.sync_copy(acc_sh.at[pl.ds(sid * _RPT, _RPT)],
                    out_hbm.at[cid, pl.ds(sid * _RPT, _RPT)])


_spmm_call = pl.kernel(
    _spmm_body,
    out_type=jax.ShapeDtypeStruct((NC, N_ALL, DIM), jnp.float32),
    mesh=_SC_MESH,
    compiler_params=_SC_PARAMS,
    scratch_types=[
        pltpu.VMEM((_CE,), jnp.int32),
        pltpu.VMEM((_CE,), jnp.int32),
        pltpu.VMEM((_CE,), jnp.float32),
        pltpu.VMEM((_CE, DIM), jnp.float32),
        pltpu.VMEM((_ZROWS, DIM), jnp.float32),
        pltpu.VMEM_SHARED((N_ALL, DIM), jnp.float32),
        pltpu.SemaphoreType.DMA,
    ],
)


def _make_gather(n_rows, chunk):
    rpt = n_rows // NW
    nchunk = rpt // chunk
    assert rpt % chunk == 0 and chunk % 8 == 0

    def _body(table_hbm, idx_hbm, out_hbm, idx_v, rows_v, sem):
        wid = lax.axis_index("c") * NS + lax.axis_index("s")
        base = wid * rpt

        def _ck(ci, carry):
            rb = base + ci * chunk
            pltpu.sync_copy(idx_hbm.at[pl.ds(rb, chunk)], idx_v)
            pltpu.async_copy(table_hbm.at[idx_v], rows_v, sem).wait()
            pltpu.sync_copy(rows_v, out_hbm.at[pl.ds(rb, chunk)])
            return carry
        lax.fori_loop(0, nchunk, _ck, 0)

    return pl.kernel(
        _body,
        out_type=jax.ShapeDtypeStruct((n_rows, DIM), jnp.float32),
        mesh=_SC_MESH,
        compiler_params=_SC_PARAMS,
        scratch_types=[
            pltpu.VMEM((chunk,), jnp.int32),
            pltpu.VMEM((chunk, DIM), jnp.float32),
            pltpu.SemaphoreType.DMA,
        ],
    )


_EGATHER_ROWS = 671744
_entity_gather = _make_gather(_EGATHER_ROWS, 512)
_batch_gather = _make_gather(2 * B, 256)


_BG = 512


def _att_kernel(h_ref, r_ref, t_ref, rel_ref, w1_ref, w2_ref, w3_ref,
                att_ref, hmean_ref):
    h3 = h_ref[...]
    t3 = t_ref[...]
    r2 = r_ref[...]
    w1a = w1_ref[0:DIM, :]
    w1b = w1_ref[DIM:2 * DIM, :]
    rproj_tab = jnp.dot(rel_ref[...], w1b, preferred_element_type=jnp.float32)
    oh3 = (r2[:, :, None] ==
           lax.broadcasted_iota(jnp.int32, (1, 1, N_REL), 2)).astype(jnp.float32)
    oh2 = oh3.reshape(_BG * T, N_REL)
    rproj = jnp.dot(oh2, rproj_tab, preferred_element_type=jnp.float32)
    h2 = h3.reshape(_BG * T, DIM)
    x = jnp.maximum(jnp.dot(h2, w1a, preferred_element_type=jnp.float32) + rproj, 0.0)
    x = jnp.maximum(jnp.dot(x, w2_ref[...], preferred_element_type=jnp.float32), 0.0)
    x3 = x.reshape(_BG, T, DIM)
    w3row = w3_ref[...].reshape(1, DIM)
    cols = []
    for t in range(T):
        st = jnp.sum(x3[:, t, :] * w3row, axis=1, keepdims=True)
        cols.append(st)
    s = jnp.concatenate(cols, axis=1)
    w = jax.nn.sigmoid(s)
    w = jnp.exp(w)
    w = w / jnp.sum(w, axis=1, keepdims=True)
    acc = jnp.zeros((_BG, DIM), jnp.float32)
    hsum = jnp.zeros((_BG, DIM), jnp.float32)
    for t in range(T):
        acc = acc + w[:, t:t + 1] * t3[:, t, :]
        hsum = hsum + h3[:, t, :]
    att_ref[...] = acc
    hmean_ref[...] = hsum * (1.0 / T)


def _att_call(h3, r2, t3, rel, w1, w2, w3):
    grid = (B // _BG,)
    return pl.pallas_call(
        _att_kernel,
        grid=grid,
        in_specs=[
            pl.BlockSpec((_BG, T, DIM), lambda i: (i, 0, 0)),
            pl.BlockSpec((_BG, T), lambda i: (i, 0)),
            pl.BlockSpec((_BG, T, DIM), lambda i: (i, 0, 0)),
            pl.BlockSpec((N_REL, DIM), lambda i: (0, 0)),
            pl.BlockSpec((2 * DIM, DIM), lambda i: (0, 0)),
            pl.BlockSpec((DIM, DIM), lambda i: (0, 0)),
            pl.BlockSpec((DIM, 1), lambda i: (0, 0)),
        ],
        out_specs=[
            pl.BlockSpec((_BG, DIM), lambda i: (i, 0)),
            pl.BlockSpec((_BG, DIM), lambda i: (i, 0)),
        ],
        out_shape=[
            jax.ShapeDtypeStruct((B, DIM), jnp.float32),
            jax.ShapeDtypeStruct((B, DIM), jnp.float32),
        ],
    )(h3, r2, t3, rel, w1, w2, w3)


def _closs_kernel(a_ref, bfull_ref, bblk_ref, l_ref, l1_ref):
    i = pl.program_id(0)
    a = a_ref[...]
    bf = bfull_ref[...]
    bb = bblk_ref[...]
    an = a / (jnp.sqrt(jnp.sum(a * a, axis=1, keepdims=True)) + 1e-8)
    bn = bf / (jnp.sqrt(jnp.sum(bf * bf, axis=1, keepdims=True)) + 1e-8)
    bnb = bb / (jnp.sqrt(jnp.sum(bb * bb, axis=1, keepdims=True)) + 1e-8)
    logits = lax.dot_general(an, bn, (((1,), (1,)), ((), ())),
                             preferred_element_type=jnp.float32) * (1.0 / C_TEMP)
    m = jnp.max(logits, axis=1, keepdims=True)
    lse = jnp.log(jnp.sum(jnp.exp(logits - m), axis=1, keepdims=True)) + m
    pos = jnp.sum(an * bnb, axis=1, keepdims=True) * (1.0 / C_TEMP)
    lblk = jnp.sum(lse - pos)
    z = jnp.sum(a * bb, axis=1, keepdims=True)
    l1blk = jnp.sum(jnp.maximum(-z, 0.0) + jnp.log(1.0 + jnp.exp(-jnp.abs(z))))

    @pl.when(i == 0)
    def _init():
        l_ref[...] = jnp.zeros((1, 1), jnp.float32)
        l1_ref[...] = jnp.zeros((1, 1), jnp.float32)

    l_ref[...] = l_ref[...] + lblk
    l1_ref[...] = l1_ref[...] + l1blk


def _closs_call(a, b):
    grid = (B // _BG,)
    return pl.pallas_call(
        _closs_kernel,
        grid=grid,
        in_specs=[
            pl.BlockSpec((_BG, DIM), lambda i: (i, 0)),
            pl.BlockSpec((B, DIM), lambda i: (0, 0)),
            pl.BlockSpec((_BG, DIM), lambda i: (i, 0)),
        ],
        out_specs=[
            pl.BlockSpec((1, 1), lambda i: (0, 0)),
            pl.BlockSpec((1, 1), lambda i: (0, 0)),
        ],
        out_shape=[
            jax.ShapeDtypeStruct((1, 1), jnp.float32),
            jax.ShapeDtypeStruct((1, 1), jnp.float32),
        ],
    )(a, b, b)


def _assemble_kernel(g5_ref, u1b_ref, g6_ref, att0_ref, att1_ref,
                     hmean0_ref, i1b_ref, out_ref):
    g5 = g5_ref[...]
    usum = jnp.zeros((_BG, DIM), jnp.float32)
    for t in range(T):
        usum = usum + g5[:, t, :]
    e_u = usum * (1.0 / T) + u1b_ref[...]
    e_v = (g6_ref[...] + att0_ref[...] + att1_ref[...]
           + hmean0_ref[...] + i1b_ref[...])
    out_ref[...] = jax.nn.sigmoid(jnp.sum(e_u * e_v, axis=1, keepdims=True))


def _assemble_call(g5, u1b, g6, att0, att1, hmean0, i1b):
    grid = (B // _BG,)
    vec = pl.BlockSpec((_BG, DIM), lambda i: (i, 0))
    return pl.pallas_call(
        _assemble_kernel,
        grid=grid,
        in_specs=[
            pl.BlockSpec((_BG, T, DIM), lambda i: (i, 0, 0)),
            vec, vec, vec, vec, vec, vec,
        ],
        out_specs=pl.BlockSpec((_BG, 1), lambda i: (i, 0)),
        out_shape=jax.ShapeDtypeStruct((B, 1), jnp.float32),
    )(g5, u1b, g6, att0, att1, hmean0, i1b)


def _lgcn(x0, row, col, val):
    acc = x0
    ego = x0
    for _ in range(LGCN_LAYERS):
        p = _spmm_call(ego, col, row, val)
        ego = p[0] + p[1]
        acc = acc + ego
    return acc * (1.0 / (LGCN_LAYERS + 1))


def kernel(items, users, item_idx, user_h, user_r, user_t, item_h, item_r,
           item_t, entity_emb, relation_emb, all_embed, W1, W2, W3, adj_row,
           adj_col, adj_val, adj2_row, adj2_col, adj2_val, u_adjdency):
    items = items.astype(jnp.int32)
    users = users.astype(jnp.int32)
    item_idx = item_idx.astype(jnp.int32)
    user_h = user_h.astype(jnp.int32)
    item_h = item_h.astype(jnp.int32)
    item_r = item_r.astype(jnp.int32)
    item_t = item_t.astype(jnp.int32)
    adj_row = adj_row.astype(jnp.int32)
    adj_col = adj_col.astype(jnp.int32)
    adj2_row = adj2_row.astype(jnp.int32)
    adj2_col = adj2_col.astype(jnp.int32)

    y1 = _lgcn(all_embed, adj_row, adj_col, adj_val)
    y2 = _lgcn(all_embed, adj2_row, adj2_col, adj2_val)

    bidx = jnp.concatenate([users, N_USERS + item_idx])
    g_y1 = _batch_gather(y1, bidx)
    g_y2 = _batch_gather(y2, bidx)
    u1b, i1b = g_y1[:B], g_y1[B:]
    u2b, i2b = g_y2[:B], g_y2[B:]

    eidx = jnp.concatenate([
        item_h[0].reshape(-1), item_t[0].reshape(-1),
        item_h[1].reshape(-1), item_t[1].reshape(-1),
        user_h[0].reshape(-1), items,
        jnp.zeros((_EGATHER_ROWS - 5 * B * T - B,), jnp.int32),
    ])
    eg = _entity_gather(entity_emb, eidx)
    bt = B * T
    gh0 = eg[0 * bt:1 * bt].reshape(B, T, DIM)
    gt0 = eg[1 * bt:2 * bt].reshape(B, T, DIM)
    gh1 = eg[2 * bt:3 * bt].reshape(B, T, DIM)
    gt1 = eg[3 * bt:4 * bt].reshape(B, T, DIM)
    g5 = eg[4 * bt:5 * bt].reshape(B, T, DIM)
    g6 = eg[5 * bt:5 * bt + B]

    att0, hmean0 = _att_call(gh0, item_r[0], gt0, relation_emb, W1, W2, W3)
    att1, _ = _att_call(gh1, item_r[1], gt1, relation_emb, W1, W2, W3)

    lu, l1u = _closs_call(u1b, u2b)
    li, l1i = _closs_call(i1b, i2b)
    c_loss = ((lu[0, 0] + li[0, 0]) / (2.0 * B)
              + l1u[0, 0] / B + l1i[0, 0] / B)

    scores2 = _assemble_call(g5, u1b, g6, att0, att1, hmean0, i1b)
    return (scores2.reshape(B), c_loss)


# speedup vs baseline: 4.8574x; 1.1532x over previous
import functools

import jax
import jax.numpy as jnp
from jax import lax
from jax.experimental import pallas as pl
from jax.experimental.pallas import tpu as pltpu
from jax.experimental.pallas import tpu_sc as plsc

N_USERS = 4096
N_ITEMS = 16384
N_ENTITY = 100000
N_REL = 32
DIM = 64
B = 4096
T = 32
NL = 2
NNZ = 655360
C_TEMP = 0.2
LGCN_LAYERS = 3
N_ALL = N_USERS + N_ITEMS

NC = 2
NS = 16
NW = NC * NS
LANES = 16

_SC_MESH = plsc.VectorSubcoreMesh(
    core_axis_name="c", subcore_axis_name="s", num_cores=NC, num_subcores=NS)
_SC_PARAMS = pltpu.CompilerParams(use_tc_tiling_on_sc=False)


_EPT = NNZ // NW
_CE = 320
_NCHUNK = _EPT // _CE
_RPT = N_ALL // NS
_ZROWS = 32


def _spmm_body(x_hbm, col_hbm, row_hbm, val_hbm, out_hbm,
               col_v, row_v, val_v, rows_v, zero_v, acc_sh, sem0, sem1):
    cid = lax.axis_index("c")
    sid = lax.axis_index("s")
    wid = cid * NS + sid
    gsems = (sem0, sem1)

    def _zb(i, carry):
        for j in range(DIM // LANES):
            zero_v[i, pl.ds(j * LANES, LANES)] = jnp.zeros((LANES,), jnp.float32)
        return carry
    lax.fori_loop(0, _ZROWS, _zb, 0)
    for k in range(_RPT // _ZROWS):
        pltpu.sync_copy(zero_v, acc_sh.at[pl.ds(sid * _RPT + k * _ZROWS, _ZROWS)])
    plsc.subcore_barrier()

    base_edge = wid * _EPT

    def _start(k, b):
        eb = base_edge + k * _CE
        pltpu.sync_copy(col_hbm.at[pl.ds(eb, _CE)], col_v.at[b])
        pltpu.sync_copy(row_hbm.at[pl.ds(eb, _CE)], row_v.at[b])
        pltpu.sync_copy(val_hbm.at[pl.ds(eb, _CE)], val_v.at[b])
        pltpu.async_copy(x_hbm.at[col_v.at[b]], rows_v.at[b], gsems[b])

    def _finish(k, b):
        pltpu.make_async_copy(x_hbm.at[col_v.at[b]], rows_v.at[b],
                              gsems[b]).wait()

        def _scale(g, c2):
            vv = val_v[b, pl.ds(g * LANES, LANES)]
            for j in range(LANES):
                v = vv[j]
                e = g * LANES + j
                for kk in range(DIM // LANES):
                    sl = pl.ds(kk * LANES, LANES)
                    rows_v[b, e, sl] = rows_v[b, e, sl] * v
            return c2
        lax.fori_loop(0, _CE // LANES, _scale, 0)
        pltpu.sync_copy(rows_v.at[b], acc_sh.at[row_v.at[b]], add=True)

    _start(0, 0)

    def _pair(p, carry):
        k0 = p * 2
        _start(k0 + 1, 1)
        _finish(k0, 0)

        @pl.when(k0 + 2 < _NCHUNK)
        def _():
            _start(k0 + 2, 0)
        _finish(k0 + 1, 1)
        return carry
    lax.fori_loop(0, _NCHUNK // 2, _pair, 0)

    plsc.subcore_barrier()
    pltpu.sync_copy(acc_sh.at[pl.ds(sid * _RPT, _RPT)],
                    out_hbm.at[cid, pl.ds(sid * _RPT, _RPT)])


_spmm_call = pl.kernel(
    _spmm_body,
    out_type=jax.ShapeDtypeStruct((NC, N_ALL, DIM), jnp.float32),
    mesh=_SC_MESH,
    compiler_params=_SC_PARAMS,
    scratch_types=[
        pltpu.VMEM((2, _CE), jnp.int32),
        pltpu.VMEM((2, _CE), jnp.int32),
        pltpu.VMEM((2, _CE), jnp.float32),
        pltpu.VMEM((2, _CE, DIM), jnp.float32),
        pltpu.VMEM((_ZROWS, DIM), jnp.float32),
        pltpu.VMEM_SHARED((N_ALL, DIM), jnp.float32),
        pltpu.SemaphoreType.DMA,
        pltpu.SemaphoreType.DMA,
    ],
)


def _make_gather(n_rows, chunk):
    rpt = n_rows // NW
    nchunk = rpt // chunk
    assert rpt % chunk == 0 and chunk % 8 == 0
    assert nchunk == 1 or nchunk % 2 == 0

    def _body(table_hbm, idx_hbm, out_hbm, idx_v, rows_v, sem0, sem1):
        wid = lax.axis_index("c") * NS + lax.axis_index("s")
        base = wid * rpt
        sems = (sem0, sem1)

        def _start(k, b):
            rb = base + k * chunk
            pltpu.sync_copy(idx_hbm.at[pl.ds(rb, chunk)], idx_v.at[b])
            pltpu.async_copy(table_hbm.at[idx_v.at[b]], rows_v.at[b], sems[b])

        def _finish(k, b):
            pltpu.make_async_copy(table_hbm.at[idx_v.at[b]], rows_v.at[b],
                                  sems[b]).wait()
            pltpu.sync_copy(rows_v.at[b],
                            out_hbm.at[pl.ds(base + k * chunk, chunk)])

        _start(0, 0)
        if nchunk == 1:
            _finish(0, 0)
        else:
            def _pair(p, carry):
                k0 = p * 2
                _start(k0 + 1, 1)
                _finish(k0, 0)

                @pl.when(k0 + 2 < nchunk)
                def _():
                    _start(k0 + 2, 0)
                _finish(k0 + 1, 1)
                return carry
            lax.fori_loop(0, nchunk // 2, _pair, 0)

    return pl.kernel(
        _body,
        out_type=jax.ShapeDtypeStruct((n_rows, DIM), jnp.float32),
        mesh=_SC_MESH,
        compiler_params=_SC_PARAMS,
        scratch_types=[
            pltpu.VMEM((2, chunk), jnp.int32),
            pltpu.VMEM((2, chunk, DIM), jnp.float32),
            pltpu.SemaphoreType.DMA,
            pltpu.SemaphoreType.DMA,
        ],
    )


_EGATHER_ROWS = 688128
_entity_gather = _make_gather(_EGATHER_ROWS, 512)
_batch_gather = _make_gather(2 * B, 256)


_BG = 512


def _att_kernel(h_ref, r_ref, t_ref, rel_ref, w1_ref, w2_ref, w3_ref,
                att_ref, hmean_ref):
    h3 = h_ref[...]
    t3 = t_ref[...]
    r2 = r_ref[...]
    w1a = w1_ref[0:DIM, :]
    w1b = w1_ref[DIM:2 * DIM, :]
    rproj_tab = jnp.dot(rel_ref[...], w1b, preferred_element_type=jnp.float32)
    oh3 = (r2[:, :, None] ==
           lax.broadcasted_iota(jnp.int32, (1, 1, N_REL), 2)).astype(jnp.float32)
    oh2 = oh3.reshape(_BG * T, N_REL)
    rproj = jnp.dot(oh2, rproj_tab, preferred_element_type=jnp.float32)
    h2 = h3.reshape(_BG * T, DIM)
    x = jnp.maximum(jnp.dot(h2, w1a, preferred_element_type=jnp.float32) + rproj, 0.0)
    x = jnp.maximum(jnp.dot(x, w2_ref[...], preferred_element_type=jnp.float32), 0.0)
    x3 = x.reshape(_BG, T, DIM)
    w3row = w3_ref[...].reshape(1, DIM)
    cols = []
    for t in range(T):
        st = jnp.sum(x3[:, t, :] * w3row, axis=1, keepdims=True)
        cols.append(st)
    s = jnp.concatenate(cols, axis=1)
    w = jax.nn.sigmoid(s)
    w = jnp.exp(w)
    w = w / jnp.sum(w, axis=1, keepdims=True)
    acc = jnp.zeros((_BG, DIM), jnp.float32)
    hsum = jnp.zeros((_BG, DIM), jnp.float32)
    for t in range(T):
        acc = acc + w[:, t:t + 1] * t3[:, t, :]
        hsum = hsum + h3[:, t, :]
    att_ref[...] = acc
    hmean_ref[...] = hsum * (1.0 / T)


def _att_call(h3, r2, t3, rel, w1, w2, w3):
    grid = (B // _BG,)
    return pl.pallas_call(
        _att_kernel,
        grid=grid,
        in_specs=[
            pl.BlockSpec((_BG, T, DIM), lambda i: (i, 0, 0)),
            pl.BlockSpec((_BG, T), lambda i: (i, 0)),
            pl.BlockSpec((_BG, T, DIM), lambda i: (i, 0, 0)),
            pl.BlockSpec((N_REL, DIM), lambda i: (0, 0)),
            pl.BlockSpec((2 * DIM, DIM), lambda i: (0, 0)),
            pl.BlockSpec((DIM, DIM), lambda i: (0, 0)),
            pl.BlockSpec((DIM, 1), lambda i: (0, 0)),
        ],
        out_specs=[
            pl.BlockSpec((_BG, DIM), lambda i: (i, 0)),
            pl.BlockSpec((_BG, DIM), lambda i: (i, 0)),
        ],
        out_shape=[
            jax.ShapeDtypeStruct((B, DIM), jnp.float32),
            jax.ShapeDtypeStruct((B, DIM), jnp.float32),
        ],
    )(h3, r2, t3, rel, w1, w2, w3)


def _closs_kernel(a_ref, bfull_ref, bblk_ref, l_ref, l1_ref):
    i = pl.program_id(0)
    a = a_ref[...]
    bf = bfull_ref[...]
    bb = bblk_ref[...]
    an = a / (jnp.sqrt(jnp.sum(a * a, axis=1, keepdims=True)) + 1e-8)
    bn = bf / (jnp.sqrt(jnp.sum(bf * bf, axis=1, keepdims=True)) + 1e-8)
    bnb = bb / (jnp.sqrt(jnp.sum(bb * bb, axis=1, keepdims=True)) + 1e-8)
    logits = lax.dot_general(an, bn, (((1,), (1,)), ((), ())),
                             preferred_element_type=jnp.float32) * (1.0 / C_TEMP)
    m = jnp.max(logits, axis=1, keepdims=True)
    lse = jnp.log(jnp.sum(jnp.exp(logits - m), axis=1, keepdims=True)) + m
    pos = jnp.sum(an * bnb, axis=1, keepdims=True) * (1.0 / C_TEMP)
    lblk = jnp.sum(lse - pos)
    z = jnp.sum(a * bb, axis=1, keepdims=True)
    l1blk = jnp.sum(jnp.maximum(-z, 0.0) + jnp.log(1.0 + jnp.exp(-jnp.abs(z))))

    @pl.when(i == 0)
    def _init():
        l_ref[...] = jnp.zeros((1, 1), jnp.float32)
        l1_ref[...] = jnp.zeros((1, 1), jnp.float32)

    l_ref[...] = l_ref[...] + lblk
    l1_ref[...] = l1_ref[...] + l1blk


def _closs_call(a, b):
    grid = (B // _BG,)
    return pl.pallas_call(
        _closs_kernel,
        grid=grid,
        in_specs=[
            pl.BlockSpec((_BG, DIM), lambda i: (i, 0)),
            pl.BlockSpec((B, DIM), lambda i: (0, 0)),
            pl.BlockSpec((_BG, DIM), lambda i: (i, 0)),
        ],
        out_specs=[
            pl.BlockSpec((1, 1), lambda i: (0, 0)),
            pl.BlockSpec((1, 1), lambda i: (0, 0)),
        ],
        out_shape=[
            jax.ShapeDtypeStruct((1, 1), jnp.float32),
            jax.ShapeDtypeStruct((1, 1), jnp.float32),
        ],
    )(a, b, b)


def _assemble_kernel(g5_ref, u1b_ref, g6_ref, att0_ref, att1_ref,
                     hmean0_ref, i1b_ref, out_ref):
    g5 = g5_ref[...]
    usum = jnp.zeros((_BG, DIM), jnp.float32)
    for t in range(T):
        usum = usum + g5[:, t, :]
    e_u = usum * (1.0 / T) + u1b_ref[...]
    e_v = (g6_ref[...] + att0_ref[...] + att1_ref[...]
           + hmean0_ref[...] + i1b_ref[...])
    out_ref[...] = jax.nn.sigmoid(jnp.sum(e_u * e_v, axis=1, keepdims=True))


def _assemble_call(g5, u1b, g6, att0, att1, hmean0, i1b):
    grid = (B // _BG,)
    vec = pl.BlockSpec((_BG, DIM), lambda i: (i, 0))
    return pl.pallas_call(
        _assemble_kernel,
        grid=grid,
        in_specs=[
            pl.BlockSpec((_BG, T, DIM), lambda i: (i, 0, 0)),
            vec, vec, vec, vec, vec, vec,
        ],
        out_specs=pl.BlockSpec((_BG, 1), lambda i: (i, 0)),
        out_shape=jax.ShapeDtypeStruct((B, 1), jnp.float32),
    )(g5, u1b, g6, att0, att1, hmean0, i1b)


def _lgcn(x0, row, col, val):
    acc = x0
    ego = x0
    for _ in range(LGCN_LAYERS):
        p = _spmm_call(ego, col, row, val)
        ego = p[0] + p[1]
        acc = acc + ego
    return acc * (1.0 / (LGCN_LAYERS + 1))


def kernel(items, users, item_idx, user_h, user_r, user_t, item_h, item_r,
           item_t, entity_emb, relation_emb, all_embed, W1, W2, W3, adj_row,
           adj_col, adj_val, adj2_row, adj2_col, adj2_val, u_adjdency):
    items = items.astype(jnp.int32)
    users = users.astype(jnp.int32)
    item_idx = item_idx.astype(jnp.int32)
    user_h = user_h.astype(jnp.int32)
    item_h = item_h.astype(jnp.int32)
    item_r = item_r.astype(jnp.int32)
    item_t = item_t.astype(jnp.int32)
    adj_row = adj_row.astype(jnp.int32)
    adj_col = adj_col.astype(jnp.int32)
    adj2_row = adj2_row.astype(jnp.int32)
    adj2_col = adj2_col.astype(jnp.int32)

    y1 = _lgcn(all_embed, adj_row, adj_col, adj_val)
    y2 = _lgcn(all_embed, adj2_row, adj2_col, adj2_val)

    bidx = jnp.concatenate([users, N_USERS + item_idx])
    g_y1 = _batch_gather(y1, bidx)
    g_y2 = _batch_gather(y2, bidx)
    u1b, i1b = g_y1[:B], g_y1[B:]
    u2b, i2b = g_y2[:B], g_y2[B:]

    eidx = jnp.concatenate([
        item_h[0].reshape(-1), item_t[0].reshape(-1),
        item_h[1].reshape(-1), item_t[1].reshape(-1),
        user_h[0].reshape(-1), items,
        jnp.zeros((_EGATHER_ROWS - 5 * B * T - B,), jnp.int32),
    ])
    eg = _entity_gather(entity_emb, eidx)
    bt = B * T
    gh0 = eg[0 * bt:1 * bt].reshape(B, T, DIM)
    gt0 = eg[1 * bt:2 * bt].reshape(B, T, DIM)
    gh1 = eg[2 * bt:3 * bt].reshape(B, T, DIM)
    gt1 = eg[3 * bt:4 * bt].reshape(B, T, DIM)
    g5 = eg[4 * bt:5 * bt].reshape(B, T, DIM)
    g6 = eg[5 * bt:5 * bt + B]

    att0, hmean0 = _att_call(gh0, item_r[0], gt0, relation_emb, W1, W2, W3)
    att1, _ = _att_call(gh1, item_r[1], gt1, relation_emb, W1, W2, W3)

    lu, l1u = _closs_call(u1b, u2b)
    li, l1i = _closs_call(i1b, i2b)
    c_loss = ((lu[0, 0] + li[0, 0]) / (2.0 * B)
              + l1u[0, 0] / B + l1i[0, 0] / B)

    scores2 = _assemble_call(g5, u1b, g6, att0, att1, hmean0, i1b)
    return (scores2.reshape(B), c_loss)


# speedup vs baseline: 5.1870x; 1.0679x over previous
import functools

import jax
import jax.numpy as jnp
from jax import lax
from jax.experimental import pallas as pl
from jax.experimental.pallas import tpu as pltpu
from jax.experimental.pallas import tpu_sc as plsc

N_USERS = 4096
N_ITEMS = 16384
N_ENTITY = 100000
N_REL = 32
DIM = 64
B = 4096
T = 32
NL = 2
NNZ = 655360
C_TEMP = 0.2
LGCN_LAYERS = 3
N_ALL = N_USERS + N_ITEMS

NC = 2
NS = 16
NW = NC * NS
LANES = 16

_SC_MESH = plsc.VectorSubcoreMesh(
    core_axis_name="c", subcore_axis_name="s", num_cores=NC, num_subcores=NS)
_SC_PARAMS = pltpu.CompilerParams(use_tc_tiling_on_sc=False)


_CE = 320
_EPT = NNZ // NS
_NCHUNK = _EPT // _CE
_RPT = N_ALL // NS
_ZROWS = 32


def _spmm_body(xmult, x_hbm, col_hbm, row_hbm, val_hbm, out_hbm,
               col_v, row_v, val_v, rows_v, zero_v, acc_sh,
               gsem0, gsem1, wsem0, wsem1, zsem):
    cid = lax.axis_index("c")
    sid = lax.axis_index("s")
    gsems = (gsem0, gsem1)
    wsems = (wsem0, wsem1)
    base_edge = cid * NNZ + sid * _EPT
    coff = cid * (N_ALL * xmult)

    def _start(k, b):
        eb = base_edge + k * _CE
        pltpu.sync_copy(col_hbm.at[pl.ds(eb, _CE)], col_v.at[b])
        pltpu.sync_copy(row_hbm.at[pl.ds(eb, _CE)], row_v.at[b])
        pltpu.sync_copy(val_hbm.at[pl.ds(eb, _CE)], val_v.at[b])
        if xmult:
            def _off(g, c2):
                sl = pl.ds(g * LANES, LANES)
                col_v[b, sl] = col_v[b, sl] + coff
                return c2
            lax.fori_loop(0, _CE // LANES, _off, 0)
        pltpu.async_copy(x_hbm.at[col_v.at[b]], rows_v.at[b], gsems[b])

    def _wait_scatter(b):
        pltpu.make_async_copy(rows_v.at[b], acc_sh.at[row_v.at[b]],
                              wsems[b]).wait()

    def _finish(k, b):
        pltpu.make_async_copy(x_hbm.at[col_v.at[b]], rows_v.at[b],
                              gsems[b]).wait()

        def _scale(g, c2):
            vv = val_v[b, pl.ds(g * LANES, LANES)]
            for j in range(LANES):
                v = vv[j]
                e = g * LANES + j
                for kk in range(DIM // LANES):
                    sl = pl.ds(kk * LANES, LANES)
                    rows_v[b, e, sl] = rows_v[b, e, sl] * v
            return c2
        lax.fori_loop(0, _CE // LANES, _scale, 0)
        pltpu.async_copy(rows_v.at[b], acc_sh.at[row_v.at[b]], wsems[b],
                         add=True)

    def _zb(i, carry):
        for j in range(DIM // LANES):
            zero_v[i, pl.ds(j * LANES, LANES)] = jnp.zeros((LANES,), jnp.float32)
        return carry
    lax.fori_loop(0, _ZROWS, _zb, 0)
    _start(0, 0)
    for k in range(_RPT // _ZROWS):
        pltpu.async_copy(
            zero_v, acc_sh.at[pl.ds(sid * _RPT + k * _ZROWS, _ZROWS)], zsem)
    for k in range(_RPT // _ZROWS):
        pltpu.make_async_copy(
            zero_v, acc_sh.at[pl.ds(sid * _RPT + k * _ZROWS, _ZROWS)],
            zsem).wait()
    plsc.subcore_barrier()

    _start(1, 1)

    def _pair(p, carry):
        k0 = p * 2
        _finish(k0, 0)

        @pl.when(k0 + 2 < _NCHUNK)
        def _():
            _wait_scatter(0)
            _start(k0 + 2, 0)
        _finish(k0 + 1, 1)

        @pl.when(k0 + 3 < _NCHUNK)
        def _():
            _wait_scatter(1)
            _start(k0 + 3, 1)
        return carry
    lax.fori_loop(0, _NCHUNK // 2, _pair, 0)

    _wait_scatter(0)
    _wait_scatter(1)
    plsc.subcore_barrier()
    pltpu.sync_copy(acc_sh.at[pl.ds(sid * _RPT, _RPT)],
                    out_hbm.at[cid, pl.ds(sid * _RPT, _RPT)])


def _make_spmm(xmult):
    return pl.kernel(
        functools.partial(_spmm_body, xmult),
        out_type=jax.ShapeDtypeStruct((NC, N_ALL, DIM), jnp.float32),
        mesh=_SC_MESH,
        compiler_params=_SC_PARAMS,
        scratch_types=[
            pltpu.VMEM((2, _CE), jnp.int32),
            pltpu.VMEM((2, _CE), jnp.int32),
            pltpu.VMEM((2, _CE), jnp.float32),
            pltpu.VMEM((2, _CE, DIM), jnp.float32),
            pltpu.VMEM((_ZROWS, DIM), jnp.float32),
            pltpu.VMEM_SHARED((N_ALL, DIM), jnp.float32),
            pltpu.SemaphoreType.DMA,
            pltpu.SemaphoreType.DMA,
            pltpu.SemaphoreType.DMA,
            pltpu.SemaphoreType.DMA,
            pltpu.SemaphoreType.DMA,
        ],
    )


_spmm_first = _make_spmm(0)
_spmm_next = _make_spmm(1)


def _make_gather(n_rows, chunk):
    rpt = n_rows // NW
    nchunk = rpt // chunk
    assert rpt % chunk == 0 and chunk % 8 == 0
    assert nchunk == 1 or nchunk % 2 == 0

    def _body(table_hbm, idx_hbm, out_hbm, idx_v, rows_v,
              gsem0, gsem1, wsem0, wsem1):
        wid = lax.axis_index("c") * NS + lax.axis_index("s")
        base = wid * rpt
        gsems = (gsem0, gsem1)
        wsems = (wsem0, wsem1)

        def _out_ref(k, b):
            return out_hbm.at[pl.ds(base + k * chunk, chunk)]

        def _start(k, b):
            rb = base + k * chunk
            pltpu.sync_copy(idx_hbm.at[pl.ds(rb, chunk)], idx_v.at[b])
            pltpu.async_copy(table_hbm.at[idx_v.at[b]], rows_v.at[b], gsems[b])

        def _finish(k, b):
            pltpu.make_async_copy(table_hbm.at[idx_v.at[b]], rows_v.at[b],
                                  gsems[b]).wait()
            pltpu.async_copy(rows_v.at[b], _out_ref(k, b), wsems[b])

        def _wait_out(k, b):
            pltpu.make_async_copy(rows_v.at[b], _out_ref(k, b), wsems[b]).wait()

        _start(0, 0)
        if nchunk == 1:
            _finish(0, 0)
            _wait_out(0, 0)
        else:
            _start(1, 1)

            def _pair(p, carry):
                k0 = p * 2
                _finish(k0, 0)

                @pl.when(k0 + 2 < nchunk)
                def _():
                    _wait_out(k0, 0)
                    _start(k0 + 2, 0)
                _finish(k0 + 1, 1)

                @pl.when(k0 + 3 < nchunk)
                def _():
                    _wait_out(k0 + 1, 1)
                    _start(k0 + 3, 1)
                return carry
            lax.fori_loop(0, nchunk // 2, _pair, 0)
            _wait_out(nchunk - 2, 0)
            _wait_out(nchunk - 1, 1)

    return pl.kernel(
        _body,
        out_type=jax.ShapeDtypeStruct((n_rows, DIM), jnp.float32),
        mesh=_SC_MESH,
        compiler_params=_SC_PARAMS,
        scratch_types=[
            pltpu.VMEM((2, chunk), jnp.int32),
            pltpu.VMEM((2, chunk, DIM), jnp.float32),
            pltpu.SemaphoreType.DMA,
            pltpu.SemaphoreType.DMA,
            pltpu.SemaphoreType.DMA,
            pltpu.SemaphoreType.DMA,
        ],
    )


_EGATHER_ROWS = 688128
_entity_gather = _make_gather(_EGATHER_ROWS, 512)


_GS_CHUNK = 128
_GS_RPT = 2 * B // NS
_GS_NCHUNK = _GS_RPT // _GS_CHUNK


def _gsum_body(x0_hbm, y1_hbm, y2_hbm, y3_hbm, idx_hbm, out_hbm,
               idx_v, idx2_v, rows_v, out_v, sem0, sem1, sem2, sem3, wsem):
    cid = lax.axis_index("c")
    sid = lax.axis_index("s")
    base = sid * _GS_RPT
    coff = cid * N_ALL
    sems = (sem0, sem1, sem2, sem3)

    def _chunk(k, carry):
        rb = base + k * _GS_CHUNK
        pltpu.sync_copy(idx_hbm.at[pl.ds(rb, _GS_CHUNK)], idx_v)

        def _off(g, c2):
            sl = pl.ds(g * LANES, LANES)
            idx2_v[sl] = idx_v[sl] + coff
            return c2
        lax.fori_loop(0, _GS_CHUNK // LANES, _off, 0)

        pltpu.async_copy(x0_hbm.at[idx_v], rows_v.at[0], sems[0])
        pltpu.async_copy(y1_hbm.at[idx2_v], rows_v.at[1], sems[1])
        pltpu.async_copy(y2_hbm.at[idx2_v], rows_v.at[2], sems[2])
        pltpu.async_copy(y3_hbm.at[idx2_v], rows_v.at[3], sems[3])
        pltpu.make_async_copy(x0_hbm.at[idx_v], rows_v.at[0], sems[0]).wait()
        pltpu.make_async_copy(y1_hbm.at[idx2_v], rows_v.at[1], sems[1]).wait()
        pltpu.make_async_copy(y2_hbm.at[idx2_v], rows_v.at[2], sems[2]).wait()
        pltpu.make_async_copy(y3_hbm.at[idx2_v], rows_v.at[3], sems[3]).wait()

        def _sum(g, c2):
            e = g // (DIM // LANES)
            kk = g % (DIM // LANES)
            sl = pl.ds(kk * LANES, LANES)
            out_v[e, sl] = (rows_v[0, e, sl] + rows_v[1, e, sl]
                            + rows_v[2, e, sl] + rows_v[3, e, sl]) * 0.25
            return c2
        lax.fori_loop(0, _GS_CHUNK * (DIM // LANES), _sum, 0)
        pltpu.sync_copy(out_v, out_hbm.at[cid, pl.ds(rb, _GS_CHUNK)])
        return carry
    lax.fori_loop(0, _GS_NCHUNK, _chunk, 0)


_gsum_call = pl.kernel(
    _gsum_body,
    out_type=jax.ShapeDtypeStruct((NC, 2 * B, DIM), jnp.float32),
    mesh=_SC_MESH,
    compiler_params=_SC_PARAMS,
    scratch_types=[
        pltpu.VMEM((_GS_CHUNK,), jnp.int32),
        pltpu.VMEM((_GS_CHUNK,), jnp.int32),
        pltpu.VMEM((4, _GS_CHUNK, DIM), jnp.float32),
        pltpu.VMEM((_GS_CHUNK, DIM), jnp.float32),
        pltpu.SemaphoreType.DMA,
        pltpu.SemaphoreType.DMA,
        pltpu.SemaphoreType.DMA,
        pltpu.SemaphoreType.DMA,
        pltpu.SemaphoreType.DMA,
    ],
)


_BG = 512


def _att_kernel(h_ref, r_ref, t_ref, rel_ref, w1_ref, w2_ref, w3_ref,
                att_ref, hmean_ref):
    h3 = h_ref[...]
    t3 = t_ref[...]
    r2 = r_ref[...]
    w1a = w1_ref[0:DIM, :]
    w1b = w1_ref[DIM:2 * DIM, :]
    rproj_tab = jnp.dot(rel_ref[...], w1b, preferred_element_type=jnp.float32)
    oh3 = (r2[:, :, None] ==
           lax.broadcasted_iota(jnp.int32, (1, 1, N_REL), 2)).astype(jnp.float32)
    oh2 = oh3.reshape(_BG * T, N_REL)
    rproj = jnp.dot(oh2, rproj_tab, preferred_element_type=jnp.float32)
    h2 = h3.reshape(_BG * T, DIM)
    x = jnp.maximum(jnp.dot(h2, w1a, preferred_element_type=jnp.float32) + rproj, 0.0)
    x = jnp.maximum(jnp.dot(x, w2_ref[...], preferred_element_type=jnp.float32), 0.0)
    x3 = x.reshape(_BG, T, DIM)
    w3row = w3_ref[...].reshape(1, DIM)
    cols = []
    for t in range(T):
        st = jnp.sum(x3[:, t, :] * w3row, axis=1, keepdims=True)
        cols.append(st)
    s = jnp.concatenate(cols, axis=1)
    w = jax.nn.sigmoid(s)
    w = jnp.exp(w)
    w = w / jnp.sum(w, axis=1, keepdims=True)
    acc = jnp.zeros((_BG, DIM), jnp.float32)
    hsum = jnp.zeros((_BG, DIM), jnp.float32)
    for t in range(T):
        acc = acc + w[:, t:t + 1] * t3[:, t, :]
        hsum = hsum + h3[:, t, :]
    att_ref[...] = acc
    hmean_ref[...] = hsum * (1.0 / T)


def _att_call(h3, r2, t3, rel, w1, w2, w3):
    grid = (B // _BG,)
    return pl.pallas_call(
        _att_kernel,
        grid=grid,
        in_specs=[
            pl.BlockSpec((_BG, T, DIM), lambda i: (i, 0, 0)),
            pl.BlockSpec((_BG, T), lambda i: (i, 0)),
            pl.BlockSpec((_BG, T, DIM), lambda i: (i, 0, 0)),
            pl.BlockSpec((N_REL, DIM), lambda i: (0, 0)),
            pl.BlockSpec((2 * DIM, DIM), lambda i: (0, 0)),
            pl.BlockSpec((DIM, DIM), lambda i: (0, 0)),
            pl.BlockSpec((DIM, 1), lambda i: (0, 0)),
        ],
        out_specs=[
            pl.BlockSpec((_BG, DIM), lambda i: (i, 0)),
            pl.BlockSpec((_BG, DIM), lambda i: (i, 0)),
        ],
        out_shape=[
            jax.ShapeDtypeStruct((B, DIM), jnp.float32),
            jax.ShapeDtypeStruct((B, DIM), jnp.float32),
        ],
    )(h3, r2, t3, rel, w1, w2, w3)


def _closs_kernel(a_ref, bfull_ref, bblk_ref, l_ref, l1_ref):
    i = pl.program_id(0)
    a = a_ref[...]
    bf = bfull_ref[...]
    bb = bblk_ref[...]
    an = a / (jnp.sqrt(jnp.sum(a * a, axis=1, keepdims=True)) + 1e-8)
    bn = bf / (jnp.sqrt(jnp.sum(bf * bf, axis=1, keepdims=True)) + 1e-8)
    bnb = bb / (jnp.sqrt(jnp.sum(bb * bb, axis=1, keepdims=True)) + 1e-8)
    logits = lax.dot_general(an, bn, (((1,), (1,)), ((), ())),
                             preferred_element_type=jnp.float32) * (1.0 / C_TEMP)
    m = jnp.max(logits, axis=1, keepdims=True)
    lse = jnp.log(jnp.sum(jnp.exp(logits - m), axis=1, keepdims=True)) + m
    pos = jnp.sum(an * bnb, axis=1, keepdims=True) * (1.0 / C_TEMP)
    lblk = jnp.sum(lse - pos)
    z = jnp.sum(a * bb, axis=1, keepdims=True)
    l1blk = jnp.sum(jnp.maximum(-z, 0.0) + jnp.log(1.0 + jnp.exp(-jnp.abs(z))))

    @pl.when(i == 0)
    def _init():
        l_ref[...] = jnp.zeros((1, 1), jnp.float32)
        l1_ref[...] = jnp.zeros((1, 1), jnp.float32)

    l_ref[...] = l_ref[...] + lblk
    l1_ref[...] = l1_ref[...] + l1blk


def _closs_call(a, b):
    grid = (B // _BG,)
    return pl.pallas_call(
        _closs_kernel,
        grid=grid,
        in_specs=[
            pl.BlockSpec((_BG, DIM), lambda i: (i, 0)),
            pl.BlockSpec((B, DIM), lambda i: (0, 0)),
            pl.BlockSpec((_BG, DIM), lambda i: (i, 0)),
        ],
        out_specs=[
            pl.BlockSpec((1, 1), lambda i: (0, 0)),
            pl.BlockSpec((1, 1), lambda i: (0, 0)),
        ],
        out_shape=[
            jax.ShapeDtypeStruct((1, 1), jnp.float32),
            jax.ShapeDtypeStruct((1, 1), jnp.float32),
        ],
    )(a, b, b)


def _assemble_kernel(g5_ref, u1b_ref, g6_ref, att0_ref, att1_ref,
                     hmean0_ref, i1b_ref, out_ref):
    g5 = g5_ref[...]
    usum = jnp.zeros((_BG, DIM), jnp.float32)
    for t in range(T):
        usum = usum + g5[:, t, :]
    e_u = usum * (1.0 / T) + u1b_ref[...]
    e_v = (g6_ref[...] + att0_ref[...] + att1_ref[...]
           + hmean0_ref[...] + i1b_ref[...])
    out_ref[...] = jax.nn.sigmoid(jnp.sum(e_u * e_v, axis=1, keepdims=True))


def _assemble_call(g5, u1b, g6, att0, att1, hmean0, i1b):
    grid = (B // _BG,)
    vec = pl.BlockSpec((_BG, DIM), lambda i: (i, 0))
    return pl.pallas_call(
        _assemble_kernel,
        grid=grid,
        in_specs=[
            pl.BlockSpec((_BG, T, DIM), lambda i: (i, 0, 0)),
            vec, vec, vec, vec, vec, vec,
        ],
        out_specs=pl.BlockSpec((_BG, 1), lambda i: (i, 0)),
        out_shape=jax.ShapeDtypeStruct((B, 1), jnp.float32),
    )(g5, u1b, g6, att0, att1, hmean0, i1b)


def kernel(items, users, item_idx, user_h, user_r, user_t, item_h, item_r,
           item_t, entity_emb, relation_emb, all_embed, W1, W2, W3, adj_row,
           adj_col, adj_val, adj2_row, adj2_col, adj2_val, u_adjdency):
    items = items.astype(jnp.int32)
    users = users.astype(jnp.int32)
    item_idx = item_idx.astype(jnp.int32)
    user_h = user_h.astype(jnp.int32)
    item_h = item_h.astype(jnp.int32)
    item_r = item_r.astype(jnp.int32)
    item_t = item_t.astype(jnp.int32)
    adj_row = adj_row.astype(jnp.int32)
    adj_col = adj_col.astype(jnp.int32)
    adj2_row = adj2_row.astype(jnp.int32)
    adj2_col = adj2_col.astype(jnp.int32)

    eidx = jnp.concatenate([
        item_h[0].reshape(-1), item_t[0].reshape(-1),
        item_h[1].reshape(-1), item_t[1].reshape(-1),
        user_h[0].reshape(-1), items,
        jnp.zeros((_EGATHER_ROWS - 5 * B * T - B,), jnp.int32),
    ])
    eg = _entity_gather(entity_emb, eidx)
    bt = B * T
    gh0 = eg[0 * bt:1 * bt].reshape(B, T, DIM)
    gt0 = eg[1 * bt:2 * bt].reshape(B, T, DIM)
    gh1 = eg[2 * bt:3 * bt].reshape(B, T, DIM)
    gt1 = eg[3 * bt:4 * bt].reshape(B, T, DIM)
    g5 = eg[4 * bt:5 * bt].reshape(B, T, DIM)
    g6 = eg[5 * bt:5 * bt + B]

    att0, hmean0 = _att_call(gh0, item_r[0], gt0, relation_emb, W1, W2, W3)
    att1, _ = _att_call(gh1, item_r[1], gt1, relation_emb, W1, W2, W3)

    col2 = jnp.concatenate([adj_col, adj2_col])
    row2 = jnp.concatenate([adj_row, adj2_row])
    val2 = jnp.concatenate([adj_val, adj2_val])
    y1 = _spmm_first(all_embed, col2, row2, val2)
    y2 = _spmm_next(y1.reshape(2 * N_ALL, DIM), col2, row2, val2)
    y3 = _spmm_next(y2.reshape(2 * N_ALL, DIM), col2, row2, val2)

    bidx = jnp.concatenate([users, N_USERS + item_idx])
    gs = _gsum_call(all_embed, y1.reshape(2 * N_ALL, DIM),
                    y2.reshape(2 * N_ALL, DIM), y3.reshape(2 * N_ALL, DIM),
                    bidx)
    u1b, i1b = gs[0, :B], gs[0, B:]
    u2b, i2b = gs[1, :B], gs[1, B:]

    lu, l1u = _closs_call(u1b, u2b)
    li, l1i = _closs_call(i1b, i2b)
    c_loss = ((lu[0, 0] + li[0, 0]) / (2.0 * B)
              + l1u[0, 0] / B + l1i[0, 0] / B)

    scores2 = _assemble_call(g5, u1b, g6, att0, att1, hmean0, i1b)
    return (scores2.reshape(B), c_loss)
